# baseline (device time: 20370 ns/iter reference)
import jax
import jax.numpy as jnp
from jax import lax
from jax.experimental import pallas as pl
from jax.experimental.pallas import tpu as pltpu

N_DEV = 8
B, SQ, DM = 2, 128, 512
HQ, DH = 4, 64
DQK = HQ * DH
SKV_SHARD = 128
WINDOW = 128
N_SRC = 2

_MESH = pltpu.DeviceIdType.MESH
_CompilerParams = getattr(pltpu, "CompilerParams", None) or getattr(
    pltpu, "TPUCompilerParams"
)


def kernel(x, Wq, K_ext, V_ext, Wo):
    def body(x_ref, wq_ref, k_ref, v_ref, wo_ref, out_ref,
             kv_buf, send_sems, recv_sems):
        my = lax.axis_index("i")

        barrier = pltpu.get_barrier_semaphore()
        for t in range(N_DEV):
            @pl.when(my != t)
            def _sig(t=t):
                pltpu.semaphore_signal(
                    barrier, 1, device_id=(t,), device_id_type=_MESH
                )
        pltpu.semaphore_wait(barrier, N_DEV - 1)

        for s in range(N_SRC):
            @pl.when(my == s)
            def _stage(s=s):
                kv_buf[s, 0] = (
                    k_ref[...].reshape(B, SKV_SHARD, DQK).astype(jnp.bfloat16)
                )
                kv_buf[s, 1] = (
                    v_ref[...].reshape(B, SKV_SHARD, DQK).astype(jnp.bfloat16)
                )

        def send_rdmas(s):
            rdmas = []
            for t in range(N_DEV):
                if t == s:
                    continue
                rdmas.append(pltpu.make_async_remote_copy(
                    src_ref=kv_buf.at[s],
                    dst_ref=kv_buf.at[s],
                    send_sem=send_sems.at[t],
                    recv_sem=recv_sems.at[s],
                    device_id=(t,),
                    device_id_type=_MESH,
                ))
            return rdmas

        for s in range(N_SRC):
            @pl.when(my == s)
            def _send(s=s):
                for r in send_rdmas(s):
                    r.start()

        for s in range(N_SRC):
            @pl.when(my != s)
            def _recv(s=s):
                r = pltpu.make_async_remote_copy(
                    src_ref=kv_buf.at[s],
                    dst_ref=kv_buf.at[s],
                    send_sem=send_sems.at[s],
                    recv_sem=recv_sems.at[s],
                    device_id=(s,),
                    device_id_type=_MESH,
                )
                r.wait_recv()

        skv = N_SRC * SKV_SHARD
        kfull = jnp.concatenate([kv_buf[0, 0], kv_buf[1, 0]], axis=1)
        vfull = jnp.concatenate([kv_buf[0, 1], kv_buf[1, 1]], axis=1)

        qi = lax.broadcasted_iota(jnp.int32, (SQ, skv), 0)
        ki = lax.broadcasted_iota(jnp.int32, (SQ, skv), 1)
        mask = jnp.abs(qi - ki) <= WINDOW

        wq = wq_ref[...].astype(jnp.bfloat16)
        wo = wo_ref[...].astype(jnp.bfloat16)
        for b in range(B):
            xb = x_ref[b].astype(jnp.bfloat16)
            qb = lax.dot_general(
                xb, wq, (((1,), (0,)), ((), ())),
                preferred_element_type=jnp.float32,
            ).astype(jnp.bfloat16)
            ctx_parts = []
            for h in range(HQ):
                qh = qb[:, h * DH:(h + 1) * DH]
                kh = kfull[b, :, h * DH:(h + 1) * DH]
                s = lax.dot_general(
                    qh, kh, (((1,), (1,)), ((), ())),
                    preferred_element_type=jnp.float32,
                ) * 0.125
                s = jnp.where(mask, s, -1e9)
                m = jnp.max(s, axis=1, keepdims=True)
                p = jnp.exp(s - m)
                w = (p / jnp.sum(p, axis=1, keepdims=True)).astype(jnp.bfloat16)
                vh = vfull[b, :, h * DH:(h + 1) * DH]
                ctx_parts.append(lax.dot_general(
                    w, vh, (((1,), (0,)), ((), ())),
                    preferred_element_type=jnp.float32,
                ))
            ctx = jnp.concatenate(ctx_parts, axis=1).astype(jnp.bfloat16)
            out_ref[b] = lax.dot_general(
                ctx, wo, (((1,), (0,)), ((), ())),
                preferred_element_type=jnp.float32,
            )

        for s in range(N_SRC):
            @pl.when(my == s)
            def _drain(s=s):
                for r in send_rdmas(s):
                    r.wait_send()

    return pl.pallas_call(
        body,
        out_shape=jax.ShapeDtypeStruct((B, SQ, DM), jnp.float32),
        in_specs=[pl.BlockSpec(memory_space=pltpu.VMEM)] * 5,
        out_specs=pl.BlockSpec(memory_space=pltpu.VMEM),
        scratch_shapes=[
            pltpu.VMEM((N_SRC, 2, B, SKV_SHARD, DQK), jnp.bfloat16),
            pltpu.SemaphoreType.DMA((N_DEV,)),
            pltpu.SemaphoreType.DMA((N_SRC,)),
        ],
        compiler_params=_CompilerParams(collective_id=0),
    )(x, Wq, K_ext, V_ext, Wo)


# device time: 19016 ns/iter; 1.0712x vs baseline; 1.0712x over previous
import jax
import jax.numpy as jnp
from jax import lax
from jax.experimental import pallas as pl
from jax.experimental.pallas import tpu as pltpu

N_DEV = 8
B, SQ, DM = 2, 128, 512
HQ, DH = 4, 64
DQK = HQ * DH
SKV_SHARD = 128
WINDOW = 128
N_SRC = 2
PCOLS = DQK + HQ

_MESH = pltpu.DeviceIdType.MESH
_CompilerParams = getattr(pltpu, "CompilerParams", None) or getattr(
    pltpu, "TPUCompilerParams"
)


def kernel(x, Wq, K_ext, V_ext, Wo):
    def body(x_ref, wq_ref, k_ref, v_ref, wo_ref, out_ref,
             pbuf, send_sems, recv_sems):
        my = lax.axis_index("i")

        for s in range(N_SRC):
            @pl.when(my != s)
            def _sig(s=s):
                pltpu.semaphore_signal(
                    pltpu.get_barrier_semaphore(), 1,
                    device_id=(s,), device_id_type=_MESH,
                )
        for s in range(N_SRC):
            @pl.when(my == s)
            def _bar():
                pltpu.semaphore_wait(
                    pltpu.get_barrier_semaphore(), N_DEV - 1
                )

        for s in range(N_SRC):
            @pl.when(my == s)
            def _partial(s=s):
                wq = wq_ref[...].astype(jnp.bfloat16)
                k = k_ref[...].reshape(B, SKV_SHARD, DQK).astype(jnp.bfloat16)
                v = v_ref[...].reshape(B, SKV_SHARD, DQK)
                for b in range(B):
                    xb = x_ref[b].astype(jnp.bfloat16)
                    qb = lax.dot_general(
                        xb, wq, (((1,), (0,)), ((), ())),
                        preferred_element_type=jnp.float32,
                    ).astype(jnp.bfloat16)
                    for h in range(HQ):
                        qh = qb[:, h * DH:(h + 1) * DH]
                        kh = k[b, :, h * DH:(h + 1) * DH]
                        sc = lax.dot_general(
                            qh, kh, (((1,), (1,)), ((), ())),
                            preferred_element_type=jnp.float32,
                        ) * 0.125
                        p = jnp.exp(sc)
                        if s == 1:
                            qi = lax.broadcasted_iota(
                                jnp.int32, (SQ, SKV_SHARD), 0)
                            ji = lax.broadcasted_iota(
                                jnp.int32, (SQ, SKV_SHARD), 1)
                            p = jnp.where(ji <= qi, p, 0.0)
                        vh = jnp.concatenate(
                            [v[b, :, h * DH:(h + 1) * DH],
                             jnp.ones((SKV_SHARD, 1), jnp.float32)],
                            axis=1,
                        ).astype(jnp.bfloat16)
                        cl = lax.dot_general(
                            p.astype(jnp.bfloat16), vh,
                            (((1,), (0,)), ((), ())),
                            preferred_element_type=jnp.float32,
                        )
                        pbuf[s, b, :, h * DH:(h + 1) * DH] = (
                            cl[:, :DH].astype(jnp.bfloat16))
                        pbuf[s, b, :, DQK + h:DQK + h + 1] = (
                            cl[:, DH:DH + 1].astype(jnp.bfloat16))

        def send_rdmas(s):
            rdmas = []
            for t in range(N_DEV):
                if t == s:
                    continue
                rdmas.append(pltpu.make_async_remote_copy(
                    src_ref=pbuf.at[s],
                    dst_ref=pbuf.at[s],
                    send_sem=send_sems.at[t],
                    recv_sem=recv_sems.at[s],
                    device_id=(t,),
                    device_id_type=_MESH,
                ))
            return rdmas

        for s in range(N_SRC):
            @pl.when(my == s)
            def _send(s=s):
                for r in send_rdmas(s):
                    r.start()

        for s in range(N_SRC):
            @pl.when(my != s)
            def _recv(s=s):
                r = pltpu.make_async_remote_copy(
                    src_ref=pbuf.at[s],
                    dst_ref=pbuf.at[s],
                    send_sem=send_sems.at[s],
                    recv_sem=recv_sems.at[s],
                    device_id=(s,),
                    device_id_type=_MESH,
                )
                r.wait_recv()

        wo = wo_ref[...].astype(jnp.bfloat16)
        for b in range(B):
            tot = (pbuf[0, b].astype(jnp.float32)
                   + pbuf[1, b].astype(jnp.float32))
            denom = jnp.concatenate(
                [jnp.broadcast_to(tot[:, DQK + h:DQK + h + 1], (SQ, DH))
                 for h in range(HQ)],
                axis=1,
            )
            ctx = (tot[:, :DQK] / denom).astype(jnp.bfloat16)
            out_ref[b] = lax.dot_general(
                ctx, wo, (((1,), (0,)), ((), ())),
                preferred_element_type=jnp.float32,
            )

        for s in range(N_SRC):
            @pl.when(my == s)
            def _drain(s=s):
                for r in send_rdmas(s):
                    r.wait_send()

    return pl.pallas_call(
        body,
        out_shape=jax.ShapeDtypeStruct((B, SQ, DM), jnp.float32),
        in_specs=[pl.BlockSpec(memory_space=pltpu.VMEM)] * 5,
        out_specs=pl.BlockSpec(memory_space=pltpu.VMEM),
        scratch_shapes=[
            pltpu.VMEM((N_SRC, B, SQ, PCOLS), jnp.bfloat16),
            pltpu.SemaphoreType.DMA((N_DEV,)),
            pltpu.SemaphoreType.DMA((N_SRC,)),
        ],
        compiler_params=_CompilerParams(collective_id=0),
    )(x, Wq, K_ext, V_ext, Wo)


# device time: 13771 ns/iter; 1.4792x vs baseline; 1.3809x over previous
import jax
import jax.numpy as jnp
from jax import lax
from jax.experimental import pallas as pl
from jax.experimental.pallas import tpu as pltpu

N_DEV = 8
B, SQ, DM = 2, 128, 512
HQ, DH = 4, 64
DQK = HQ * DH
SKV_SHARD = 128
WINDOW = 128
N_SRC = 2
PCOLS = DQK + HQ

_MESH = pltpu.DeviceIdType.MESH
_CompilerParams = getattr(pltpu, "CompilerParams", None) or getattr(
    pltpu, "TPUCompilerParams"
)

CHILDREN = {
    0: {0: [1, 3, 4], 1: [4]},
    1: {1: [0, 2, 5], 0: [5]},
    2: {1: [3, 6]},
    3: {0: [2, 7]},
    4: {0: [6]},
    5: {1: [7]},
    6: {},
    7: {},
}
PARENT = {
    0: {1: 1},
    1: {0: 0},
    2: {0: 3, 1: 1},
    3: {0: 0, 1: 2},
    4: {0: 0, 1: 0},
    5: {0: 1, 1: 1},
    6: {0: 4, 1: 2},
    7: {0: 3, 1: 5},
}


def _plan(p):
    plan = []
    if p < N_SRC:
        for t in CHILDREN[p].get(p, []):
            plan.append((p, t, len(plan)))
    for c in _recv_order(p):
        if p < N_SRC and c == p:
            continue
        for t in CHILDREN[p].get(c, []):
            plan.append((c, t, len(plan)))
    return plan


def _recv_order(p):
    cs = sorted(PARENT[p].keys())
    return sorted(cs, key=lambda c: -len(CHILDREN[p].get(c, [])))


def kernel(x, Wq, K_ext, V_ext, Wo):
    def body(x_ref, wq_ref, k_ref, v_ref, wo_ref, out_ref,
             pbuf, send_sems, recv_sems):
        my = lax.axis_index("i")

        def mk(c, t, slot):
            return pltpu.make_async_remote_copy(
                src_ref=pbuf.at[c],
                dst_ref=pbuf.at[c],
                send_sem=send_sems.at[slot],
                recv_sem=recv_sems.at[c],
                device_id=(t,),
                device_id_type=_MESH,
            )

        def compute_partial(s):
            wq = wq_ref[...].astype(jnp.bfloat16)
            k = k_ref[...].reshape(B, SKV_SHARD, DQK).astype(jnp.bfloat16)
            v = v_ref[...].reshape(B, SKV_SHARD, DQK)
            for b in range(B):
                xb = x_ref[b].astype(jnp.bfloat16)
                qb = lax.dot_general(
                    xb, wq, (((1,), (0,)), ((), ())),
                    preferred_element_type=jnp.float32,
                ).astype(jnp.bfloat16)
                for h in range(HQ):
                    qh = qb[:, h * DH:(h + 1) * DH]
                    kh = k[b, :, h * DH:(h + 1) * DH]
                    sc = lax.dot_general(
                        qh, kh, (((1,), (1,)), ((), ())),
                        preferred_element_type=jnp.float32,
                    ) * 0.125
                    p = jnp.exp(sc)
                    if s == 1:
                        qi = lax.broadcasted_iota(
                            jnp.int32, (SQ, SKV_SHARD), 0)
                        ji = lax.broadcasted_iota(
                            jnp.int32, (SQ, SKV_SHARD), 1)
                        p = jnp.where(ji <= qi, p, 0.0)
                    vh = jnp.concatenate(
                        [v[b, :, h * DH:(h + 1) * DH],
                         jnp.ones((SKV_SHARD, 1), jnp.float32)],
                        axis=1,
                    ).astype(jnp.bfloat16)
                    cl = lax.dot_general(
                        p.astype(jnp.bfloat16), vh,
                        (((1,), (0,)), ((), ())),
                        preferred_element_type=jnp.float32,
                    )
                    pbuf[s, b, :, h * DH:(h + 1) * DH] = (
                        cl[:, :DH].astype(jnp.bfloat16))
                    pbuf[s, b, :, DQK + h:DQK + h + 1] = (
                        cl[:, DH:DH + 1].astype(jnp.bfloat16))

        for p in range(N_DEV):
            @pl.when(my == p)
            def _node(p=p):
                for c in sorted(PARENT[p]):
                    pltpu.semaphore_signal(
                        pltpu.get_barrier_semaphore(), 1,
                        device_id=(PARENT[p][c],), device_id_type=_MESH,
                    )
                if p < N_SRC:
                    compute_partial(p)
                n_edges = sum(len(v) for v in CHILDREN[p].values())
                if n_edges:
                    pltpu.semaphore_wait(
                        pltpu.get_barrier_semaphore(), n_edges)
                if p < N_SRC:
                    for c, t, slot in _plan(p):
                        if c == p:
                            mk(c, t, slot).start()
                for c in _recv_order(p):
                    mk(c, PARENT[p][c], 0).wait_recv()
                    for cc, t, slot in _plan(p):
                        if cc == c and not (p < N_SRC and c == p):
                            mk(cc, t, slot).start()

        wo = wo_ref[...].astype(jnp.bfloat16)
        for b in range(B):
            tot = (pbuf[0, b].astype(jnp.float32)
                   + pbuf[1, b].astype(jnp.float32))
            denom = jnp.concatenate(
                [jnp.broadcast_to(tot[:, DQK + h:DQK + h + 1], (SQ, DH))
                 for h in range(HQ)],
                axis=1,
            )
            ctx = (tot[:, :DQK] / denom).astype(jnp.bfloat16)
            out_ref[b] = lax.dot_general(
                ctx, wo, (((1,), (0,)), ((), ())),
                preferred_element_type=jnp.float32,
            )

        for p in range(N_DEV):
            if not CHILDREN[p]:
                continue

            @pl.when(my == p)
            def _drain(p=p):
                for c, t, slot in _plan(p):
                    mk(c, t, slot).wait_send()

    return pl.pallas_call(
        body,
        out_shape=jax.ShapeDtypeStruct((B, SQ, DM), jnp.float32),
        in_specs=[pl.BlockSpec(memory_space=pltpu.VMEM)] * 5,
        out_specs=pl.BlockSpec(memory_space=pltpu.VMEM),
        scratch_shapes=[
            pltpu.VMEM((N_SRC, B, SQ, PCOLS), jnp.bfloat16),
            pltpu.SemaphoreType.DMA((6,)),
            pltpu.SemaphoreType.DMA((N_SRC,)),
        ],
        compiler_params=_CompilerParams(collective_id=0),
    )(x, Wq, K_ext, V_ext, Wo)


# device time: 10225 ns/iter; 1.9922x vs baseline; 1.3468x over previous
import jax
import jax.numpy as jnp
from jax import lax
from jax.experimental import pallas as pl
from jax.experimental.pallas import tpu as pltpu

N_DEV = 8
B, SQ, DM = 2, 128, 512
HQ, DH = 4, 64
DQK = HQ * DH
SKV_SHARD = 128
WINDOW = 128
N_SRC = 2
PCOLS = DQK + HQ

_MESH = pltpu.DeviceIdType.MESH
_CompilerParams = getattr(pltpu, "CompilerParams", None) or getattr(
    pltpu, "TPUCompilerParams"
)

CHILDREN = {
    0: {0: [1, 3, 4]},
    1: {1: [0, 2, 5]},
    2: {1: [3, 6]},
    3: {0: [2, 7]},
    4: {0: [6, 5]},
    5: {1: [4, 7]},
    6: {},
    7: {},
}
PARENT = {
    0: {1: 1},
    1: {0: 0},
    2: {0: 3, 1: 1},
    3: {0: 0, 1: 2},
    4: {0: 0, 1: 5},
    5: {0: 4, 1: 1},
    6: {0: 4, 1: 2},
    7: {0: 3, 1: 5},
}


def _plan(p):
    plan = []
    if p < N_SRC:
        for b in range(B):
            for t in CHILDREN[p].get(p, []):
                plan.append((p, b, t, len(plan)))
    for c in _recv_order(p):
        if p < N_SRC and c == p:
            continue
        for b in range(B):
            for t in CHILDREN[p].get(c, []):
                plan.append((c, b, t, len(plan)))
    return plan


def _recv_order(p):
    cs = sorted(PARENT[p].keys())
    return sorted(cs, key=lambda c: -len(CHILDREN[p].get(c, [])))


def kernel(x, Wq, K_ext, V_ext, Wo):
    def body(x_hbm, wq_hbm, k_hbm, v_hbm, wo_hbm, out_hbm,
             pbuf, send_sems, recv_sems,
             x_ref, wq_ref, k_ref, v_ref, wo_ref, dma_sems,
             out_vmem, out_sems):
        my = lax.axis_index("i")

        _dma = [(x_hbm, x_ref), (wq_hbm, wq_ref), (k_hbm, k_ref),
                (v_hbm, v_ref), (wo_hbm, wo_ref)]

        def dma(i):
            return pltpu.make_async_copy(_dma[i][0], _dma[i][1],
                                         dma_sems.at[i])

        dma(4).start()

        @pl.when(my < N_SRC)
        def _fetch():
            for i in range(4):
                dma(i).start()

        def mk(c, b, t, slot):
            return pltpu.make_async_remote_copy(
                src_ref=pbuf.at[c, b],
                dst_ref=pbuf.at[c, b],
                send_sem=send_sems.at[slot],
                recv_sem=recv_sems.at[c, b],
                device_id=(t,),
                device_id_type=_MESH,
            )

        def compute_partial(s, b, wait_kv=False):
            wq = wq_ref[...].astype(jnp.bfloat16)
            xb = x_ref[b].astype(jnp.bfloat16)
            qb = lax.dot_general(
                xb, wq, (((1,), (0,)), ((), ())),
                preferred_element_type=jnp.float32,
            ).astype(jnp.bfloat16)
            if wait_kv:
                dma(2).wait()
                dma(3).wait()
            vbt = jnp.swapaxes(v_ref[b], 1, 2).astype(jnp.bfloat16)
            ones_col = jnp.ones((SKV_SHARD, 1), jnp.bfloat16)
            for h in range(HQ):
                qh = qb[:, h * DH:(h + 1) * DH]
                kh = k_ref[b, h].astype(jnp.bfloat16)
                sc = lax.dot_general(
                    qh, kh, (((1,), (0,)), ((), ())),
                    preferred_element_type=jnp.float32,
                ) * 0.125
                p = jnp.exp(sc)
                if s == 1:
                    qi = lax.broadcasted_iota(
                        jnp.int32, (SQ, SKV_SHARD), 0)
                    ji = lax.broadcasted_iota(
                        jnp.int32, (SQ, SKV_SHARD), 1)
                    p = jnp.where(ji <= qi, p, 0.0)
                vh = jnp.concatenate([vbt[h], ones_col], axis=1)
                cl = lax.dot_general(
                    p.astype(jnp.bfloat16), vh,
                    (((1,), (0,)), ((), ())),
                    preferred_element_type=jnp.float32,
                )
                pbuf[s, b, :, h * DH:(h + 1) * DH] = (
                    cl[:, :DH].astype(jnp.bfloat16))
                pbuf[s, b, :, DQK + h:DQK + h + 1] = (
                    cl[:, DH:DH + 1].astype(jnp.bfloat16))

        def combine_b(b):
            wo = wo_ref[...].astype(jnp.bfloat16)
            tot = (pbuf[0, b].astype(jnp.float32)
                   + pbuf[1, b].astype(jnp.float32))
            denom = jnp.concatenate(
                [jnp.broadcast_to(tot[:, DQK + h:DQK + h + 1], (SQ, DH))
                 for h in range(HQ)],
                axis=1,
            )
            ctx = (tot[:, :DQK] / denom).astype(jnp.bfloat16)
            out_vmem[b] = lax.dot_general(
                ctx, wo, (((1,), (0,)), ((), ())),
                preferred_element_type=jnp.float32,
            ).astype(jnp.bfloat16)
            pltpu.make_async_copy(
                out_vmem.at[b], out_hbm.at[b], out_sems.at[b]).start()

        def start_fwds(p, c, b):
            for cc, bb, t, slot in _plan(p):
                if cc == c and bb == b and not (p < N_SRC and c == p):
                    mk(cc, bb, t, slot).start()

        for p in range(N_DEV):
            @pl.when(my == p)
            def _node(p=p):
                for c in sorted(PARENT[p]):
                    pltpu.semaphore_signal(
                        pltpu.get_barrier_semaphore(), 1,
                        device_id=(PARENT[p][c],), device_id_type=_MESH,
                    )
                n_edges = sum(len(v) for v in CHILDREN[p].values())
                if p < N_SRC:
                    dma(0).wait()
                    dma(1).wait()
                    for b in range(B):
                        compute_partial(p, b, wait_kv=(b == 0))
                        if b == 0 and n_edges:
                            pltpu.semaphore_wait(
                                pltpu.get_barrier_semaphore(), n_edges)
                        for c, bb, t, slot in _plan(p):
                            if c == p and bb == b:
                                mk(c, bb, t, slot).start()
                    (c_o,) = _recv_order(p)
                    dma(4).wait()
                    for b in range(B):
                        mk(c_o, b, PARENT[p][c_o], 0).wait_recv()
                        start_fwds(p, c_o, b)
                        combine_b(b)
                else:
                    if n_edges:
                        pltpu.semaphore_wait(
                            pltpu.get_barrier_semaphore(), n_edges)
                    ro = _recv_order(p)
                    fwd_cs = [c for c in ro if CHILDREN[p].get(c)]
                    rest = [c for c in ro if not CHILDREN[p].get(c)]
                    for c in fwd_cs:
                        for b in range(B):
                            mk(c, b, PARENT[p][c], 0).wait_recv()
                            start_fwds(p, c, b)
                    dma(4).wait()
                    for b in range(B):
                        for c in rest:
                            mk(c, b, PARENT[p][c], 0).wait_recv()
                        combine_b(b)

        for p in range(N_DEV):
            if not CHILDREN[p]:
                continue

            @pl.when(my == p)
            def _drain(p=p):
                for c, b, t, slot in _plan(p):
                    mk(c, b, t, slot).wait_send()

        for b in range(B):
            pltpu.make_async_copy(
                out_vmem.at[b], out_hbm.at[b], out_sems.at[b]).wait()

    k2 = K_ext.transpose(0, 2, 3, 1)
    v2 = V_ext.transpose(0, 2, 3, 1)
    hbm = pltpu.MemorySpace.HBM
    x, Wq, Wo = (pltpu.with_memory_space_constraint(a, hbm)
                 for a in (x, Wq, Wo))
    k2 = pltpu.with_memory_space_constraint(k2, hbm)
    v2 = pltpu.with_memory_space_constraint(v2, hbm)
    return pl.pallas_call(
        body,
        out_shape=jax.ShapeDtypeStruct((B, SQ, DM), jnp.bfloat16),
        in_specs=[pl.BlockSpec(memory_space=pl.ANY)] * 5,
        out_specs=pl.BlockSpec(memory_space=pltpu.MemorySpace.HBM),
        scratch_shapes=[
            pltpu.VMEM((N_SRC, B, SQ, PCOLS), jnp.bfloat16),
            pltpu.SemaphoreType.DMA((8,)),
            pltpu.SemaphoreType.DMA((N_SRC, B)),
            pltpu.VMEM((B, SQ, DM), jnp.float32),
            pltpu.VMEM((DM, DQK), jnp.float32),
            pltpu.VMEM((B, HQ, DH, SKV_SHARD), jnp.float32),
            pltpu.VMEM((B, HQ, DH, SKV_SHARD), jnp.float32),
            pltpu.VMEM((DQK, DM), jnp.float32),
            pltpu.SemaphoreType.DMA((5,)),
            pltpu.VMEM((B, SQ, DM), jnp.bfloat16),
            pltpu.SemaphoreType.DMA((B,)),
        ],
        compiler_params=_CompilerParams(collective_id=0),
    )(x, Wq, k2, v2, Wo)
